# baseline (speedup 1.0000x reference)
import functools

import jax
import jax.numpy as jnp
from jax import lax
from jax.experimental import pallas as pl
from jax.experimental.pallas import tpu as pltpu
from jax.experimental.pallas import tpu_sc as plsc

NC = 2
NS = 16
NW = NC * NS

B = 16384
D = 64
HID = 32
ACT = 18

SUB = 8
NT = 1000000 // SUB
LANES = 16

ROWS_W = B // NW
CH = 128
NCH = ROWS_W // CH
NG = CH // LANES


@functools.cache
def _make_sc_gather():
    mesh = plsc.VectorSubcoreMesh(
        core_axis_name="c", subcore_axis_name="s", num_cores=NC, num_subcores=NS
    )

    @functools.partial(
        pl.kernel,
        out_type=jax.ShapeDtypeStruct((NW, NCH, CH, 128), jnp.float32),
        mesh=mesh,
        scratch_types=[
            pltpu.VMEM((NCH, CH), jnp.int32),
            pltpu.VMEM((2, LANES, SUB, D), jnp.float32),
            pltpu.VMEM((CH, 128), jnp.float32),
            pltpu.SemaphoreType.DMA,
            pltpu.SemaphoreType.DMA,
        ],
        compiler_params=pltpu.CompilerParams(needs_layout_passes=False),
    )
    def _sc_gather(idx_hbm, table_hbm, out_hbm, idx_v, slabs, x_v, semA, semB):
        wid = lax.axis_index("s") * NC + lax.axis_index("c")
        pltpu.sync_copy(idx_hbm.at[wid], idx_v)
        iota = lax.iota(jnp.int32, LANES)
        d16 = [jnp.full((LANES,), d, jnp.int32) for d in range(D)]
        sems = (semA, semB)

        def extract(g, svec, descs):
            for cp in descs:
                cp.wait()
            slab = slabs.at[g % 2]
            base = g * LANES + iota
            for d in range(D):
                vals = plsc.load_gather(slab, [iota, svec, d16[d]])
                plsc.store_scatter(x_v, [base, d16[d]], vals)

        def chunk2(c, carry):
            prev = None
            for g in range(NG):
                vec = idx_v[c, pl.ds(g * LANES, LANES)]
                tvec = vec >> 3
                svec = vec & 7
                descs = []
                for r in range(LANES):
                    t = lax.reduce_max(
                        jnp.where(iota == r, tvec, -1), axes=(0,)
                    )
                    descs.append(
                        pltpu.async_copy(
                            table_hbm.at[pl.ds(t * SUB, SUB)],
                            slabs.at[g % 2, r],
                            sems[g % 2],
                        )
                    )
                if prev is not None:
                    extract(*prev)
                prev = (g, svec, descs)
            extract(*prev)
            pltpu.sync_copy(x_v, out_hbm.at[wid, c])
            return carry

        lax.fori_loop(0, NCH, chunk2, 0)

    return _sc_gather


def _mlp_body(x_ref, w1_ref, b1_ref, w2_ref, b2_ref, y_ref):
    x = x_ref[...]
    h = lax.dot_general(
        x, w1_ref[...], (((1,), (1,)), ((), ())),
        preferred_element_type=jnp.float32,
    )
    h = jnp.maximum(h + b1_ref[...], 0.0)
    y = lax.dot_general(
        h, w2_ref[...], (((1,), (1,)), ((), ())),
        preferred_element_type=jnp.float32,
    )
    y_ref[...] = y + b2_ref[...]


def _mlp(x, W1, b1, W2, b2):
    BB = 2048
    grid = (B // BB,)
    return pl.pallas_call(
        _mlp_body,
        grid=grid,
        in_specs=[
            pl.BlockSpec((BB, D), lambda i: (i, 0)),
            pl.BlockSpec((HID, D), lambda i: (0, 0)),
            pl.BlockSpec((1, HID), lambda i: (0, 0)),
            pl.BlockSpec((ACT, HID), lambda i: (0, 0)),
            pl.BlockSpec((1, ACT), lambda i: (0, 0)),
        ],
        out_specs=pl.BlockSpec((BB, ACT), lambda i: (i, 0)),
        out_shape=jax.ShapeDtypeStruct((B, ACT), jnp.float32),
    )(x, W1, b1.reshape(1, HID), W2, b2.reshape(1, ACT))


def kernel(states, table, W1, b1, W2, b2):
    idx = states.reshape(NW, NCH, CH)
    x128 = _make_sc_gather()(idx, table).reshape(B, 128)
    x = x128[:, :D]
    y = _mlp(x, W1, b1, W2, b2)
    return (y, x)


# speedup vs baseline: 1.4284x; 1.4284x over previous
import functools

import jax
import jax.numpy as jnp
from jax import lax
from jax.experimental import pallas as pl
from jax.experimental.pallas import tpu as pltpu
from jax.experimental.pallas import tpu_sc as plsc

NC = 2
NS = 16
NW = NC * NS

B = 16384
D = 64
HID = 32
ACT = 18

SUB = 8
NT = 1000000 // SUB
LANES = 16

ROWS_W = B // NW
CH = 128
NCH = ROWS_W // CH
NG = CH // LANES


@functools.cache
def _make_sc_gather():
    mesh = plsc.VectorSubcoreMesh(
        core_axis_name="c", subcore_axis_name="s", num_cores=NC, num_subcores=NS
    )

    @functools.partial(
        pl.kernel,
        out_type=jax.ShapeDtypeStruct((NW, NCH, CH, 128), jnp.float32),
        mesh=mesh,
        scratch_types=[
            pltpu.VMEM((NCH, CH), jnp.int32),
            pltpu.VMEM((2, LANES, SUB, D), jnp.float32),
            pltpu.VMEM((CH, 128), jnp.float32),
            pltpu.SemaphoreType.DMA,
            pltpu.SemaphoreType.DMA,
        ],
        compiler_params=pltpu.CompilerParams(needs_layout_passes=False),
    )
    def _sc_gather(idx_hbm, table_hbm, out_hbm, idx_v, slabs, x_v, semA, semB):
        wid = lax.axis_index("s") * NC + lax.axis_index("c")
        pltpu.sync_copy(idx_hbm.at[wid], idx_v)
        iota = lax.iota(jnp.int32, LANES)
        d16 = [jnp.full((LANES,), d, jnp.int32) for d in range(D)]
        sems = (semA, semB)

        def extract(g, svec, descs):
            for cp in descs:
                cp.wait()
            slab = slabs.at[g % 2]
            base = g * LANES + iota
            for d in range(D):
                vals = plsc.load_gather(slab, [iota, svec, d16[d]])
                plsc.store_scatter(x_v, [base, d16[d]], vals)

        def chunk(c, carry):
            prev = None
            for g in range(NG):
                vec = idx_v[c, pl.ds(g * LANES, LANES)]
                tvec = vec >> 3
                svec = vec & 7
                descs = []
                for r in range(LANES):
                    t = lax.reduce_max(
                        jnp.where(iota == r, tvec, -1), axes=(0,)
                    )
                    descs.append(
                        pltpu.async_copy(
                            table_hbm.at[t],
                            slabs.at[g % 2, r],
                            sems[g % 2],
                        )
                    )
                if prev is not None:
                    extract(*prev)
                prev = (g, svec, descs)
            extract(*prev)
            pltpu.sync_copy(x_v, out_hbm.at[wid, c])
            return carry

        lax.fori_loop(0, NCH, chunk, 0)

    return _sc_gather


def _mlp_body(x128_ref, w1_ref, b1_ref, w2_ref, b2_ref, eye_ref, yT_ref, xT_ref):
    x = x128_ref[:, :D]
    h = lax.dot_general(
        x, w1_ref[...], (((1,), (1,)), ((), ())),
        preferred_element_type=jnp.float32,
    )
    h = jnp.maximum(h + b1_ref[...], 0.0)
    yT = lax.dot_general(
        w2_ref[...], h, (((1,), (1,)), ((), ())),
        preferred_element_type=jnp.float32,
    )
    yT_ref[...] = yT + b2_ref[...]
    xT_ref[...] = lax.dot_general(
        eye_ref[...], x, (((1,), (1,)), ((), ())),
        preferred_element_type=jnp.float32,
    )


def _mlp(x128, W1, b1, W2, b2):
    BB = 2048
    grid = (B // BB,)
    return pl.pallas_call(
        _mlp_body,
        grid=grid,
        in_specs=[
            pl.BlockSpec((BB, 128), lambda i: (i, 0)),
            pl.BlockSpec((HID, D), lambda i: (0, 0)),
            pl.BlockSpec((1, HID), lambda i: (0, 0)),
            pl.BlockSpec((ACT, HID), lambda i: (0, 0)),
            pl.BlockSpec((ACT, 1), lambda i: (0, 0)),
            pl.BlockSpec((D, D), lambda i: (0, 0)),
        ],
        out_specs=[
            pl.BlockSpec((ACT, BB), lambda i: (0, i)),
            pl.BlockSpec((D, BB), lambda i: (0, i)),
        ],
        out_shape=[
            jax.ShapeDtypeStruct((ACT, B), jnp.float32),
            jax.ShapeDtypeStruct((D, B), jnp.float32),
        ],
    )(x128, W1, b1.reshape(1, HID), W2, b2.reshape(ACT, 1), jnp.eye(D, dtype=jnp.float32))


def kernel(states, table, W1, b1, W2, b2):
    idx = states.reshape(NW, NCH, CH)
    table3 = table.reshape(NT, SUB, D)
    x128 = _make_sc_gather()(idx, table3).reshape(B, 128)
    yT, xT = _mlp(x128, W1, b1, W2, b2)
    return (yT.T, xT.T)
